# Initial kernel scaffold; baseline (speedup 1.0000x reference)
import jax
import jax.numpy as jnp
from jax.experimental import pallas as pl


def kernel(x, edge_index, W1, b1, W2, b2, Wih0, Whh0, bih0, bhh0, Wih1, Whh1, bih1, bhh1, Wc1, bc1, Wc2, bc2):
    raise NotImplementedError("write your pallas kernel here")


# speedup vs baseline: 25.2794x; 25.2794x over previous
import functools

import jax
import jax.numpy as jnp
from jax import lax
from jax.experimental import pallas as pl
from jax.experimental.pallas import tpu as pltpu
from jax.experimental.pallas import tpu_sc as plsc

N = 10000
T = 10
E = 320000
F_IN = 128
H = 64

NC, NS = 2, 16
CH = 160
EPAD = NS * CH * 128
NPAD = 10112
TPC = T // NC
RPT = NPAD // NS
BN = 400
NBLK = N // BN

_mesh = plsc.VectorSubcoreMesh(core_axis_name="c", subcore_axis_name="s",
                               num_cores=NC, num_subcores=NS)


@functools.partial(
    pl.kernel,
    out_type=jax.ShapeDtypeStruct((T * NPAD,), jnp.float32),
    mesh=_mesh,
    scratch_types=[
        pltpu.VMEM((CH, 128), jnp.int32),
        pltpu.VMEM((CH, 128), jnp.float32),
        pltpu.VMEM((640,), jnp.float32),
        pltpu.VMEM((RPT,), jnp.float32),
        pltpu.SemaphoreType.DMA,
        pltpu.VMEM_SHARED((NPAD,), jnp.float32),
    ],
)
def _deg_kernel(dst_hbm, deg_out, dstv, ones2d, zb, stage, ssem, deg_sh):
    c = lax.axis_index("c")
    s = lax.axis_index("s")

    def fill_ones(i, _):
        ones2d[i // 8, pl.ds((i % 8) * 16, 16)] = jnp.full((16,), 1.0, jnp.float32)
        return 0
    lax.fori_loop(0, CH * 8, fill_ones, 0)

    def fill_zero(i, _):
        zb[pl.ds(i * 16, 16)] = jnp.zeros((16,), jnp.float32)
        return 0
    lax.fori_loop(0, 40, fill_zero, 0)

    for i in range(TPC):
        t = c * TPC + i
        base = pl.multiple_of(s * RPT, 8)
        pltpu.sync_copy(zb.at[pl.ds(0, RPT)], deg_sh.at[pl.ds(base, RPT)])
        pltpu.sync_copy(dst_hbm.at[t, s], dstv)
        plsc.subcore_barrier()
        def fire(j, _):
            pltpu.async_copy(ones2d.at[j], deg_sh.at[dstv.at[j]], ssem, add=True)
            return 0
        lax.fori_loop(0, CH, fire, 0)

        def drain(j, _):
            pltpu.make_async_copy(ones2d.at[0], deg_sh.at[dstv.at[0]], ssem).wait()
            return 0
        lax.fori_loop(0, CH, drain, 0)
        plsc.subcore_barrier()
        off = pl.multiple_of(t * NPAD + s * RPT, 8)
        pltpu.sync_copy(deg_sh.at[pl.ds(base, RPT)], stage)
        pltpu.sync_copy(stage, deg_out.at[pl.ds(off, RPT)])


@functools.partial(
    pl.kernel,
    out_type=jax.ShapeDtypeStruct((T, NPAD, 128), jnp.float32),
    mesh=_mesh,
    scratch_types=[
        pltpu.VMEM((CH // 4, 128), jnp.int32),
        pltpu.VMEM((CH // 4, 128), jnp.int32),
        pltpu.VMEM((128, 128), jnp.float32),
        pltpu.VMEM((128, 128), jnp.float32),
        pltpu.SemaphoreType.DMA,
        pltpu.SemaphoreType.DMA,
        pltpu.VMEM_SHARED((NPAD, 128), jnp.float32),
    ],
)
def _agg_kernel(y_hbm, src_hbm, dst_hbm, agg_out,
                srcv, dstv, rows0, rows1, sem0, sem1, agg_sh):
    c = lax.axis_index("c")
    s = lax.axis_index("s")
    CH4 = CH // 4

    rows = (rows0, rows1)
    sems = (sem0, sem1)
    lens = (128, 128, 128, 128, RPT - 512)
    for i in range(TPC):
        t = c * TPC + i
        base = pl.multiple_of(s * RPT, 8)

        def fill_zero(k, _):
            rows1[k // 8, pl.ds((k % 8) * 16, 16)] = jnp.zeros((16,), jnp.float32)
            return 0
        lax.fori_loop(0, 128 * 8, fill_zero, 0)

        for q in range(5):
            ln = lens[q]
            pltpu.sync_copy(rows1.at[pl.ds(0, ln)],
                            agg_sh.at[pl.ds(base + q * 128, ln)])
        plsc.subcore_barrier()

        for hf in range(4):
            pltpu.sync_copy(src_hbm.at[t, s, pl.ds(hf * CH4, CH4)], srcv)
            pltpu.sync_copy(dst_hbm.at[t, s, pl.ds(hf * CH4, CH4)], dstv)
            pltpu.make_async_copy(y_hbm.at[srcv.at[0]], rows0, sem0).start()
            pltpu.make_async_copy(y_hbm.at[srcv.at[1]], rows1, sem1).start()

            def outer(jo, _):
                for jj in range(8):
                    j = jo * 8 + jj
                    b = jj % 2
                    pltpu.make_async_copy(y_hbm.at[srcv.at[j]], rows[b],
                                          sems[b]).wait()
                    pltpu.sync_copy(rows[b], agg_sh.at[dstv.at[j]], add=True)

                    @pl.when(j + 2 < CH4)
                    def _():
                        pltpu.make_async_copy(
                            y_hbm.at[srcv.at[j + 2]], rows[b], sems[b]).start()
                return 0
            lax.fori_loop(0, CH4 // 8, outer, 0)

        plsc.subcore_barrier()
        for q in range(5):
            ln = lens[q]
            pltpu.sync_copy(agg_sh.at[pl.ds(base + q * 128, ln)],
                            rows0.at[pl.ds(0, ln)])
            pltpu.sync_copy(rows0.at[pl.ds(0, ln)],
                            agg_out.at[t, pl.ds(base + q * 128, ln)])


def _k1_body(x_ref, deg_ref, w_ref, y_ref, dinv_ref):
    dinv = lax.rsqrt(deg_ref[0] + 1.0)
    xw = jnp.dot(x_ref[0], w_ref[...], preferred_element_type=jnp.float32)
    y_ref[0] = jnp.concatenate(
        [xw * dinv, jnp.zeros((BN, 128 - H), jnp.float32)], axis=1)
    dinv_ref[0] = dinv


_k1 = pl.pallas_call(
    _k1_body,
    grid=(T, NBLK),
    in_specs=[
        pl.BlockSpec((1, BN, F_IN), lambda t, n: (t, n, 0)),
        pl.BlockSpec((1, BN, 1), lambda t, n: (t, n, 0)),
        pl.BlockSpec((F_IN, H), lambda t, n: (0, 0)),
    ],
    out_specs=[
        pl.BlockSpec((1, BN, 128), lambda t, n: (t, n, 0)),
        pl.BlockSpec((1, BN, 1), lambda t, n: (t, n, 0)),
    ],
    out_shape=[
        jax.ShapeDtypeStruct((T, NPAD, 128), jnp.float32),
        jax.ShapeDtypeStruct((T, NPAD, 1), jnp.float32),
    ],
)


def _k2_body(agg_ref, y_ref, dinv_ref, b_ref, w_ref, out_ref):
    dinv = dinv_ref[0]
    h = jnp.maximum(
        dinv * (agg_ref[0][:, :H] + y_ref[0][:, :H]) + b_ref[...], 0.0)
    y2 = dinv * jnp.dot(h, w_ref[...], preferred_element_type=jnp.float32)
    out_ref[0] = jnp.concatenate(
        [y2, jnp.zeros((BN, 128 - H), jnp.float32)], axis=1)


_k2 = pl.pallas_call(
    _k2_body,
    grid=(T, NBLK),
    in_specs=[
        pl.BlockSpec((1, BN, 128), lambda t, n: (t, n, 0)),
        pl.BlockSpec((1, BN, 128), lambda t, n: (t, n, 0)),
        pl.BlockSpec((1, BN, 1), lambda t, n: (t, n, 0)),
        pl.BlockSpec((1, H), lambda t, n: (0, 0)),
        pl.BlockSpec((H, H), lambda t, n: (0, 0)),
    ],
    out_specs=pl.BlockSpec((1, BN, 128), lambda t, n: (t, n, 0)),
    out_shape=jax.ShapeDtypeStruct((T, NPAD, 128), jnp.float32),
)


def _k3_body(agg_ref, y_ref, dinv_ref, b_ref, out_ref):
    n = pl.program_id(1)

    @pl.when(n == 0)
    def _():
        out_ref[...] = jnp.zeros_like(out_ref)

    h = jnp.maximum(
        dinv_ref[0] * (agg_ref[0][:, :H] + y_ref[0][:, :H]) + b_ref[...], 0.0)
    out_ref[0] += jnp.sum(h, axis=0, keepdims=True)

    @pl.when(n == NBLK - 1)
    def _():
        out_ref[...] *= (1.0 / N)


_k3 = pl.pallas_call(
    _k3_body,
    grid=(T, NBLK),
    in_specs=[
        pl.BlockSpec((1, BN, 128), lambda t, n: (t, n, 0)),
        pl.BlockSpec((1, BN, 128), lambda t, n: (t, n, 0)),
        pl.BlockSpec((1, BN, 1), lambda t, n: (t, n, 0)),
        pl.BlockSpec((1, H), lambda t, n: (0, 0)),
    ],
    out_specs=pl.BlockSpec((1, 1, H), lambda t, n: (t, 0, 0)),
    out_shape=jax.ShapeDtypeStruct((T, 1, H), jnp.float32),
)


def _dotT(a, w):
    return lax.dot_general(a, w, (((1,), (1,)), ((), ())),
                           preferred_element_type=jnp.float32)


def _k4_body(emb_ref, wih0, whh0, bih0, bhh0, wih1, whh1, bih1, bhh1,
             wc1, bc1, wc2, bc2, out_ref):
    b0 = bih0[...] + bhh0[...]
    b1 = bih1[...] + bhh1[...]

    def lstm_step(xt, h, cc, wih, whh, bb):
        g = _dotT(xt, wih[...]) + _dotT(h, whh[...]) + bb
        ii = jax.nn.sigmoid(g[:, 0:H])
        ff = jax.nn.sigmoid(g[:, H:2 * H])
        gg = jnp.tanh(g[:, 2 * H:3 * H])
        oo = jax.nn.sigmoid(g[:, 3 * H:4 * H])
        c2 = ff * cc + ii * gg
        return oo * jnp.tanh(c2), c2

    def step(tt, carry):
        h0, c0, h1, c1 = carry
        xt = emb_ref[pl.ds(tt, 1), :]
        h0, c0 = lstm_step(xt, h0, c0, wih0, whh0, b0)
        h1, c1 = lstm_step(h0, h1, c1, wih1, whh1, b1)
        return (h0, c0, h1, c1)

    z = jnp.zeros((1, H), jnp.float32)
    _, _, h1, _ = lax.fori_loop(0, T, step, (z, z, z, z))
    zz = jnp.maximum(_dotT(h1, wc1[...]) + bc1[...], 0.0)
    out_ref[...] = _dotT(zz, wc2[...]) + bc2[...]


_k4 = pl.pallas_call(
    _k4_body,
    out_shape=jax.ShapeDtypeStruct((1, 2), jnp.float32),
)


def kernel(x, edge_index, W1, b1, W2, b2, Wih0, Whh0, bih0, bhh0,
           Wih1, Whh1, bih1, bhh1, Wc1, bc1, Wc2, bc2):
    src = edge_index[:, 0, :]
    dst = edge_index[:, 1, :]
    src_p = jnp.pad(src, ((0, 0), (0, EPAD - E)))
    dst_p = jnp.pad(dst, ((0, 0), (0, EPAD - E)), constant_values=N)
    offs = (jnp.arange(T, dtype=jnp.int32) * NPAD)[:, None]
    src_r = (src_p + offs).reshape(T, NS, CH, 128)
    dst_r = dst_p.reshape(T, NS, CH, 128)

    deg = _deg_kernel(dst_r)
    deg3 = deg.reshape(T, NPAD, 1)
    y1, dinv3 = _k1(x, deg3[:, :N], W1)
    agg1 = _agg_kernel(y1.reshape(T * NPAD, 128), src_r, dst_r)
    y2 = _k2(agg1, y1, dinv3, b1.reshape(1, H), W2)
    agg2 = _agg_kernel(y2.reshape(T * NPAD, 128), src_r, dst_r)
    emb = _k3(agg2, y2, dinv3, b2.reshape(1, H)).reshape(T, H)
    return _k4(emb, Wih0, Whh0, bih0.reshape(1, 4 * H), bhh0.reshape(1, 4 * H),
               Wih1, Whh1, bih1.reshape(1, 4 * H), bhh1.reshape(1, 4 * H),
               Wc1, bc1.reshape(1, H // 2), Wc2, bc2.reshape(1, 2))
